# baseline (speedup 1.0000x reference)
import functools

import jax
import jax.numpy as jnp
from jax import lax
from jax.experimental import pallas as pl
from jax.experimental.pallas import tpu as pltpu
from jax.experimental.pallas import tpu_sc as plsc

USER_COUNT = 50000
ITEM_COUNT = 50000
N_NODES = USER_COUNT + ITEM_COUNT
N_EDGES = 1600000
DIM = 32
N_LAYERS = 3

HALF = N_NODES // 2
LANES = 16
EPB = 128
ROWS_PER_CHUNK = 5
CHUNK_E = EPB * ROWS_PER_CHUNK
N_TILES = 16
E_PAD = 1638400
IDX_ROWS = E_PAD // EPB
ROWS_PER_TILE = IDX_ROWS // N_TILES
CHUNKS_PER_TILE = ROWS_PER_TILE // ROWS_PER_CHUNK
TRASH = HALF
OUT_BLK = 3128
ACC_ROWS = OUT_BLK * N_TILES
OUT_BLK_LAST = HALF - OUT_BLK * (N_TILES - 1)


_GDN = lax.GatherDimensionNumbers(
    offset_dims=(), collapsed_slice_dims=(0,), start_index_map=(0,))


def _bcast_lane(vec, lane):
    idx = jnp.full((LANES, 1), lane, jnp.int32)
    return lax.gather(vec, idx, _GDN, (1,),
                      mode=lax.GatherScatterMode.PROMISE_IN_BOUNDS)


def _prop_body(x_hbm, src_hbm, dst_hbm, w_hbm, zero_hbm, out_hbm,
               src_v, dstm_v, w_v, rows_v, acc, gsem, ssem):
    c = lax.axis_index("c")
    s = lax.axis_index("s")
    base = c * HALF

    pltpu.sync_copy(zero_hbm, acc.at[pl.ds(s * OUT_BLK, OUT_BLK)])

    plsc.subcore_barrier()

    def chunk_body(k, carry):
        row0 = s * ROWS_PER_TILE + k * ROWS_PER_CHUNK
        e0 = row0 * EPB
        pltpu.sync_copy(src_hbm.at[pl.ds(row0, ROWS_PER_CHUNK)], src_v)
        pltpu.sync_copy(dst_hbm.at[pl.ds(row0, ROWS_PER_CHUNK)], dstm_v)
        pltpu.sync_copy(w_hbm.at[pl.ds(e0, CHUNK_E)], w_v)

        def mask_row(j, cy):
            dr = dstm_v.at[j]
            sr = src_v.at[j]
            for i in range(EPB // LANES):
                d = dr[pl.ds(i * LANES, LANES)]
                sv = sr[pl.ds(i * LANES, LANES)]
                ok = (d >= base) & (d < base + HALF)
                dr[pl.ds(i * LANES, LANES)] = jnp.where(ok, d - base, TRASH)
                sr[pl.ds(i * LANES, LANES)] = jnp.where(ok, sv, 0)
            return cy
        lax.fori_loop(0, ROWS_PER_CHUNK, mask_row, 0)

        cps = [pltpu.async_copy(x_hbm.at[src_v.at[j]],
                                rows_v.at[pl.ds(j * EPB, EPB)], gsem)
               for j in range(ROWS_PER_CHUNK)]

        for j in range(ROWS_PER_CHUNK):
            cps[j].wait()

            def scale16(g, cy):
                wv = w_v[pl.ds(g * LANES, LANES)]
                for u in range(LANES):
                    wb = _bcast_lane(wv, u)
                    rr = rows_v.at[g * LANES + u]
                    rr[pl.ds(0, LANES)] = rr[pl.ds(0, LANES)] * wb
                    rr[pl.ds(LANES, LANES)] = rr[pl.ds(LANES, LANES)] * wb
                return cy
            lax.fori_loop(j * (EPB // LANES), (j + 1) * (EPB // LANES),
                          scale16, 0)

            pltpu.sync_copy(rows_v.at[pl.ds(j * EPB, EPB)],
                            acc.at[dstm_v.at[j]], add=True)
        return carry

    lax.fori_loop(0, CHUNKS_PER_TILE, chunk_body, 0)

    plsc.subcore_barrier()

    @pl.when(s < N_TILES - 1)
    def _():
        pltpu.sync_copy(acc.at[pl.ds(s * OUT_BLK, OUT_BLK)],
                        out_hbm.at[pl.ds(base + s * OUT_BLK, OUT_BLK)])

    @pl.when(s == N_TILES - 1)
    def _():
        pltpu.sync_copy(acc.at[pl.ds((N_TILES - 1) * OUT_BLK, OUT_BLK_LAST)],
                        out_hbm.at[pl.ds(base + (N_TILES - 1) * OUT_BLK, OUT_BLK_LAST)])


_propagate = functools.partial(
    pl.kernel,
    out_type=jax.ShapeDtypeStruct((N_NODES, DIM), jnp.float32),
    mesh=plsc.VectorSubcoreMesh(core_axis_name="c", subcore_axis_name="s"),
    compiler_params=pltpu.CompilerParams(use_tc_tiling_on_sc=False),
    scratch_types=[
        pltpu.VMEM((ROWS_PER_CHUNK, EPB), jnp.int32),
        pltpu.VMEM((ROWS_PER_CHUNK, EPB), jnp.int32),
        pltpu.VMEM((CHUNK_E,), jnp.float32),
        pltpu.VMEM((CHUNK_E, DIM), jnp.float32),
        pltpu.VMEM_SHARED((ACC_ROWS, DIM), jnp.float32),
        pltpu.SemaphoreType.DMA,
        pltpu.SemaphoreType.DMA,
    ],
)(_prop_body)


def _mean4_body(a_ref, b_ref, c_ref, d_ref, o_ref):
    o_ref[...] = (a_ref[...] + b_ref[...] + c_ref[...] + d_ref[...]) * 0.25


def _mean4(x0, x1, x2, x3):
    n_flat_rows = N_NODES * DIM // 128
    blk = 1000
    xs = [x.reshape(n_flat_rows, 128) for x in (x0, x1, x2, x3)]
    total = pl.pallas_call(
        _mean4_body,
        out_shape=jax.ShapeDtypeStruct((n_flat_rows, 128), jnp.float32),
        grid=(n_flat_rows // blk,),
        in_specs=[pl.BlockSpec((blk, 128), lambda i: (i, 0))] * 4,
        out_specs=pl.BlockSpec((blk, 128), lambda i: (i, 0)),
    )(*xs)
    return total.reshape(N_NODES, DIM)


def kernel(user_emb, item_emb, edge_index, edge_weight):
    x0 = jnp.concatenate([user_emb, item_emb], axis=0)
    src = edge_index[0].astype(jnp.int32)
    dst = edge_index[1].astype(jnp.int32)
    w = edge_weight.astype(jnp.float32)
    pad = E_PAD - N_EDGES
    srcp = jnp.pad(src, (0, pad)).reshape(IDX_ROWS, EPB)
    dstp = jnp.pad(dst, (0, pad)).reshape(IDX_ROWS, EPB)
    wp = jnp.pad(w, (0, pad))
    zblk = jnp.zeros((OUT_BLK, DIM), jnp.float32)

    x1 = _propagate(x0, srcp, dstp, wp, zblk)
    x2 = _propagate(x1, srcp, dstp, wp, zblk)
    x3 = _propagate(x2, srcp, dstp, wp, zblk)
    total = _mean4(x0, x1, x2, x3)
    return total[:USER_COUNT], total[USER_COUNT:]


# speedup vs baseline: 10.5551x; 10.5551x over previous
import functools

import jax
import jax.numpy as jnp
from jax import lax
from jax.experimental import pallas as pl
from jax.experimental.pallas import tpu as pltpu
from jax.experimental.pallas import tpu_sc as plsc

USER_COUNT = 50000
ITEM_COUNT = 50000
N_NODES = USER_COUNT + ITEM_COUNT
N_EDGES = 1600000
DIM = 32
N_LAYERS = 3

HALF = N_NODES // 2
LANES = 16
EPB = 128
ROWS_PER_CHUNK = 5
CHUNK_E = EPB * ROWS_PER_CHUNK
N_TILES = 16
E_PAD = 1638400
IDX_ROWS = E_PAD // EPB
ROWS_PER_TILE = IDX_ROWS // N_TILES
CHUNKS_PER_TILE = ROWS_PER_TILE // ROWS_PER_CHUNK
TRASH = HALF
OUT_BLK = 3128
ACC_ROWS = OUT_BLK * N_TILES
OUT_BLK_LAST = HALF - OUT_BLK * (N_TILES - 1)


_GDN = lax.GatherDimensionNumbers(
    offset_dims=(), collapsed_slice_dims=(0,), start_index_map=(0,))


def _bcast_lane(vec, lane):
    idx = jnp.full((LANES, 1), lane, jnp.int32)
    return lax.gather(vec, idx, _GDN, (1,),
                      mode=lax.GatherScatterMode.PROMISE_IN_BOUNDS)


def _prop_body(x_hbm, src_hbm, dst_hbm, w_hbm, zero_hbm, out_hbm,
               src_v, dstm_v, w_v, rows_v, acc, gsem, ssem):
    c = lax.axis_index("c")
    s = lax.axis_index("s")
    base = c * HALF

    pltpu.sync_copy(zero_hbm, acc.at[pl.ds(s * OUT_BLK, OUT_BLK)])

    plsc.subcore_barrier()

    def chunk_body(k, carry):
        row0 = s * ROWS_PER_TILE + k * ROWS_PER_CHUNK
        e0 = row0 * EPB
        pltpu.sync_copy(src_hbm.at[pl.ds(row0, ROWS_PER_CHUNK)], src_v)
        pltpu.sync_copy(dst_hbm.at[pl.ds(row0, ROWS_PER_CHUNK)], dstm_v)
        pltpu.sync_copy(w_hbm.at[pl.ds(e0, CHUNK_E)], w_v)

        def mask_row(j, cy):
            dr = dstm_v.at[j]
            for i in range(EPB // LANES):
                d = dr[pl.ds(i * LANES, LANES)]
                ok = (d >= base) & (d < base + HALF)
                dr[pl.ds(i * LANES, LANES)] = jnp.where(ok, d - base, TRASH)
            return cy
        lax.fori_loop(0, ROWS_PER_CHUNK, mask_row, 0)

        cps = [pltpu.async_copy(x_hbm.at[src_v.at[j]],
                                rows_v.at[pl.ds(j * EPB, EPB)], gsem)
               for j in range(ROWS_PER_CHUNK)]

        for j in range(ROWS_PER_CHUNK):
            cps[j].wait()

            def scale16(g, cy):
                wv = w_v[pl.ds(g * LANES, LANES)]
                for u in range(LANES):
                    wb = _bcast_lane(wv, u)
                    rr = rows_v.at[g * LANES + u]
                    rr[pl.ds(0, LANES)] = rr[pl.ds(0, LANES)] * wb
                    rr[pl.ds(LANES, LANES)] = rr[pl.ds(LANES, LANES)] * wb
                return cy
            lax.fori_loop(j * (EPB // LANES), (j + 1) * (EPB // LANES),
                          scale16, 0)

            pltpu.sync_copy(rows_v.at[pl.ds(j * EPB, EPB)],
                            acc.at[dstm_v.at[j]], add=True)
        return carry

    lax.fori_loop(0, CHUNKS_PER_TILE, chunk_body, 0)

    plsc.subcore_barrier()

    @pl.when(s < N_TILES - 1)
    def _():
        pltpu.sync_copy(acc.at[pl.ds(s * OUT_BLK, OUT_BLK)],
                        out_hbm.at[pl.ds(base + s * OUT_BLK, OUT_BLK)])

    @pl.when(s == N_TILES - 1)
    def _():
        pltpu.sync_copy(acc.at[pl.ds((N_TILES - 1) * OUT_BLK, OUT_BLK_LAST)],
                        out_hbm.at[pl.ds(base + (N_TILES - 1) * OUT_BLK, OUT_BLK_LAST)])


_propagate = functools.partial(
    pl.kernel,
    out_type=jax.ShapeDtypeStruct((N_NODES, DIM), jnp.float32),
    mesh=plsc.VectorSubcoreMesh(core_axis_name="c", subcore_axis_name="s"),
    compiler_params=pltpu.CompilerParams(use_tc_tiling_on_sc=False),
    scratch_types=[
        pltpu.VMEM((ROWS_PER_CHUNK, EPB), jnp.int32),
        pltpu.VMEM((ROWS_PER_CHUNK, EPB), jnp.int32),
        pltpu.VMEM((CHUNK_E,), jnp.float32),
        pltpu.VMEM((CHUNK_E, DIM), jnp.float32),
        pltpu.VMEM_SHARED((ACC_ROWS, DIM), jnp.float32),
        pltpu.SemaphoreType.DMA,
        pltpu.SemaphoreType.DMA,
    ],
)(_prop_body)


def _mean4_body(a_ref, b_ref, c_ref, d_ref, o_ref):
    o_ref[...] = (a_ref[...] + b_ref[...] + c_ref[...] + d_ref[...]) * 0.25


def _mean4(x0, x1, x2, x3):
    n_flat_rows = N_NODES * DIM // 128
    blk = 1000
    xs = [x.reshape(n_flat_rows, 128) for x in (x0, x1, x2, x3)]
    total = pl.pallas_call(
        _mean4_body,
        out_shape=jax.ShapeDtypeStruct((n_flat_rows, 128), jnp.float32),
        grid=(n_flat_rows // blk,),
        in_specs=[pl.BlockSpec((blk, 128), lambda i: (i, 0))] * 4,
        out_specs=pl.BlockSpec((blk, 128), lambda i: (i, 0)),
    )(*xs)
    return total.reshape(N_NODES, DIM)


def kernel(user_emb, item_emb, edge_index, edge_weight):
    x0 = jnp.concatenate([user_emb, item_emb], axis=0)
    src = edge_index[0].astype(jnp.int32)
    dst = edge_index[1].astype(jnp.int32)
    w = edge_weight.astype(jnp.float32)
    pad = E_PAD - N_EDGES
    srcp = jnp.pad(src, (0, pad)).reshape(IDX_ROWS, EPB)
    dstp = jnp.pad(dst, (0, pad)).reshape(IDX_ROWS, EPB)
    wp = jnp.pad(w, (0, pad))
    zblk = jnp.zeros((OUT_BLK, DIM), jnp.float32)

    x1 = _propagate(x0, srcp, dstp, wp, zblk)
    x2 = _propagate(x1, srcp, dstp, wp, zblk)
    x3 = _propagate(x2, srcp, dstp, wp, zblk)
    total = _mean4(x0, x1, x2, x3)
    return total[:USER_COUNT], total[USER_COUNT:]


# speedup vs baseline: 11.8315x; 1.1209x over previous
import functools

import jax
import jax.numpy as jnp
from jax import lax
from jax.experimental import pallas as pl
from jax.experimental.pallas import tpu as pltpu
from jax.experimental.pallas import tpu_sc as plsc

USER_COUNT = 50000
ITEM_COUNT = 50000
N_NODES = USER_COUNT + ITEM_COUNT
N_EDGES = 1600000
DIM = 32
N_LAYERS = 3

LANES = 16
HDIM = 16
N2 = 2 * N_NODES
EPB = 128
ROWS_PER_CHUNK = 5
CHUNK_E = EPB * ROWS_PER_CHUNK
N_TILES = 16
E_PAD = 1638400
IDX_ROWS = E_PAD // EPB
ROWS_PER_TILE = IDX_ROWS // N_TILES
CHUNKS_PER_TILE = ROWS_PER_TILE // ROWS_PER_CHUNK
ACC_BLK = 6256
ACC_ROWS = ACC_BLK * N_TILES
ACC_LAST = N_NODES - ACC_BLK * (N_TILES - 1)


_GDN = lax.GatherDimensionNumbers(
    offset_dims=(), collapsed_slice_dims=(0,), start_index_map=(0,))


def _bcast_lane(vec, lane):
    idx = jnp.full((LANES, 1), lane, jnp.int32)
    return lax.gather(vec, idx, _GDN, (1,),
                      mode=lax.GatherScatterMode.PROMISE_IN_BOUNDS)


def _prop_body(x_hbm, src_hbm, dst_hbm, w_hbm, zero_hbm, lo_hbm, hi_hbm,
               src_v, dst_v, w_v, rows_v, acc, gsem):
    c = lax.axis_index("c")
    s = lax.axis_index("s")

    pltpu.sync_copy(zero_hbm, acc.at[pl.ds(s * ACC_BLK, ACC_BLK)])

    plsc.subcore_barrier()

    def chunk_body(k, carry):
        row0 = s * ROWS_PER_TILE + k * ROWS_PER_CHUNK
        e0 = row0 * EPB
        pltpu.sync_copy(src_hbm.at[pl.ds(row0, ROWS_PER_CHUNK)], src_v)
        pltpu.sync_copy(dst_hbm.at[pl.ds(row0, ROWS_PER_CHUNK)], dst_v)
        pltpu.sync_copy(w_hbm.at[pl.ds(e0, CHUNK_E)], w_v)

        def adj_row(j, cy):
            sr = src_v.at[j]
            for i in range(EPB // LANES):
                sv = sr[pl.ds(i * LANES, LANES)]
                sr[pl.ds(i * LANES, LANES)] = sv + sv + c
            return cy
        lax.fori_loop(0, ROWS_PER_CHUNK, adj_row, 0)

        cps = [pltpu.async_copy(x_hbm.at[src_v.at[j]],
                                rows_v.at[pl.ds(j * EPB, EPB)], gsem)
               for j in range(ROWS_PER_CHUNK)]

        for j in range(ROWS_PER_CHUNK):
            cps[j].wait()

            def scale16(g, cy):
                wv = w_v[pl.ds(g * LANES, LANES)]
                for u in range(LANES):
                    wb = _bcast_lane(wv, u)
                    rr = rows_v.at[g * LANES + u]
                    rr[pl.ds(0, LANES)] = rr[pl.ds(0, LANES)] * wb
                return cy
            lax.fori_loop(j * (EPB // LANES), (j + 1) * (EPB // LANES),
                          scale16, 0)

            pltpu.sync_copy(rows_v.at[pl.ds(j * EPB, EPB)],
                            acc.at[dst_v.at[j]], add=True)
        return carry

    lax.fori_loop(0, CHUNKS_PER_TILE, chunk_body, 0)

    plsc.subcore_barrier()

    @pl.when(jnp.logical_and(c == 0, s < N_TILES - 1))
    def _():
        pltpu.sync_copy(acc.at[pl.ds(s * ACC_BLK, ACC_BLK)],
                        lo_hbm.at[pl.ds(s * ACC_BLK, ACC_BLK)])

    @pl.when(jnp.logical_and(c == 0, s == N_TILES - 1))
    def _():
        pltpu.sync_copy(acc.at[pl.ds((N_TILES - 1) * ACC_BLK, ACC_LAST)],
                        lo_hbm.at[pl.ds((N_TILES - 1) * ACC_BLK, ACC_LAST)])

    @pl.when(jnp.logical_and(c == 1, s < N_TILES - 1))
    def _():
        pltpu.sync_copy(acc.at[pl.ds(s * ACC_BLK, ACC_BLK)],
                        hi_hbm.at[pl.ds(s * ACC_BLK, ACC_BLK)])

    @pl.when(jnp.logical_and(c == 1, s == N_TILES - 1))
    def _():
        pltpu.sync_copy(acc.at[pl.ds((N_TILES - 1) * ACC_BLK, ACC_LAST)],
                        hi_hbm.at[pl.ds((N_TILES - 1) * ACC_BLK, ACC_LAST)])


_propagate = functools.partial(
    pl.kernel,
    out_type=[
        jax.ShapeDtypeStruct((N_NODES, HDIM), jnp.float32),
        jax.ShapeDtypeStruct((N_NODES, HDIM), jnp.float32),
    ],
    mesh=plsc.VectorSubcoreMesh(core_axis_name="c", subcore_axis_name="s"),
    compiler_params=pltpu.CompilerParams(use_tc_tiling_on_sc=False),
    scratch_types=[
        pltpu.VMEM((ROWS_PER_CHUNK, EPB), jnp.int32),
        pltpu.VMEM((ROWS_PER_CHUNK, EPB), jnp.int32),
        pltpu.VMEM((CHUNK_E,), jnp.float32),
        pltpu.VMEM((CHUNK_E, HDIM), jnp.float32),
        pltpu.VMEM_SHARED((ACC_ROWS, HDIM), jnp.float32),
        pltpu.SemaphoreType.DMA,
    ],
)(_prop_body)


def _mean4_body(a_ref, b_ref, c_ref, d_ref, o_ref):
    o_ref[...] = (a_ref[...] + b_ref[...] + c_ref[...] + d_ref[...]) * 0.25


def _mean4(x0, x1, x2, x3):
    n_flat_rows = N_NODES * DIM // 128
    blk = 1000
    xs = [x.reshape(n_flat_rows, 128) for x in (x0, x1, x2, x3)]
    total = pl.pallas_call(
        _mean4_body,
        out_shape=jax.ShapeDtypeStruct((n_flat_rows, 128), jnp.float32),
        grid=(n_flat_rows // blk,),
        in_specs=[pl.BlockSpec((blk, 128), lambda i: (i, 0))] * 4,
        out_specs=pl.BlockSpec((blk, 128), lambda i: (i, 0)),
    )(*xs)
    return total.reshape(N_NODES, DIM)


def _interleave(lo, hi):
    return jnp.stack([lo, hi], axis=1).reshape(N2, HDIM)


def kernel(user_emb, item_emb, edge_index, edge_weight):
    x0 = jnp.concatenate([user_emb, item_emb], axis=0)
    src = edge_index[0].astype(jnp.int32)
    dst = edge_index[1].astype(jnp.int32)
    w = edge_weight.astype(jnp.float32)
    pad = E_PAD - N_EDGES
    srcp = jnp.pad(src, (0, pad)).reshape(IDX_ROWS, EPB)
    dstp = jnp.pad(dst, (0, pad)).reshape(IDX_ROWS, EPB)
    wp = jnp.pad(w, (0, pad))
    zblk = jnp.zeros((ACC_BLK, HDIM), jnp.float32)

    xi0 = x0.reshape(N2, HDIM)
    lo1, hi1 = _propagate(xi0, srcp, dstp, wp, zblk)
    xi1 = _interleave(lo1, hi1)
    lo2, hi2 = _propagate(xi1, srcp, dstp, wp, zblk)
    xi2 = _interleave(lo2, hi2)
    lo3, hi3 = _propagate(xi2, srcp, dstp, wp, zblk)
    xi3 = _interleave(lo3, hi3)
    total = _mean4(xi0, xi1, xi2, xi3)
    return total[:USER_COUNT], total[USER_COUNT:]


# speedup vs baseline: 12.6183x; 1.0665x over previous
import functools

import jax
import jax.numpy as jnp
from jax import lax
from jax.experimental import pallas as pl
from jax.experimental.pallas import tpu as pltpu
from jax.experimental.pallas import tpu_sc as plsc

USER_COUNT = 50000
ITEM_COUNT = 50000
N_NODES = USER_COUNT + ITEM_COUNT
N_EDGES = 1600000
DIM = 32
N_LAYERS = 3

LANES = 16
HDIM = 16
N2 = 2 * N_NODES
EPB = 128
ROWS_PER_CHUNK = 8
CHUNK_E = EPB * ROWS_PER_CHUNK
N_TILES = 16
E_PAD = 1638400
IDX_ROWS = E_PAD // EPB
ROWS_PER_TILE = IDX_ROWS // N_TILES
CHUNKS_PER_TILE = ROWS_PER_TILE // ROWS_PER_CHUNK
ACC_BLK = 6256
ACC_ROWS = ACC_BLK * N_TILES
ACC_LAST = N_NODES - ACC_BLK * (N_TILES - 1)


_GDN = lax.GatherDimensionNumbers(
    offset_dims=(), collapsed_slice_dims=(0,), start_index_map=(0,))


def _bcast_lane(vec, lane):
    idx = jnp.full((LANES, 1), lane, jnp.int32)
    return lax.gather(vec, idx, _GDN, (1,),
                      mode=lax.GatherScatterMode.PROMISE_IN_BOUNDS)


def _prop_body(x_hbm, src_hbm, dst_hbm, w_hbm, zero_hbm, lo_hbm, hi_hbm,
               src_v, dst_v, w_v, rows_v, acc, gsem):
    c = lax.axis_index("c")
    s = lax.axis_index("s")

    pltpu.sync_copy(zero_hbm, acc.at[pl.ds(s * ACC_BLK, ACC_BLK)])

    plsc.subcore_barrier()

    def chunk_body(k, carry):
        row0 = s * ROWS_PER_TILE + k * ROWS_PER_CHUNK
        e0 = row0 * EPB
        pltpu.sync_copy(src_hbm.at[pl.ds(row0, ROWS_PER_CHUNK)], src_v)
        pltpu.sync_copy(dst_hbm.at[pl.ds(row0, ROWS_PER_CHUNK)], dst_v)
        pltpu.sync_copy(w_hbm.at[pl.ds(e0, CHUNK_E)], w_v)

        def adj_row(j, cy):
            sr = src_v.at[j]
            for i in range(EPB // LANES):
                sv = sr[pl.ds(i * LANES, LANES)]
                sr[pl.ds(i * LANES, LANES)] = sv + sv + c
            return cy
        lax.fori_loop(0, ROWS_PER_CHUNK, adj_row, 0)

        cps = [pltpu.async_copy(x_hbm.at[src_v.at[j]],
                                rows_v.at[pl.ds(j * EPB, EPB)], gsem)
               for j in range(ROWS_PER_CHUNK)]

        for j in range(ROWS_PER_CHUNK):
            cps[j].wait()

            def scale16(g, cy):
                wv = w_v[pl.ds(g * LANES, LANES)]
                for u in range(LANES):
                    wb = _bcast_lane(wv, u)
                    rr = rows_v.at[g * LANES + u]
                    rr[pl.ds(0, LANES)] = rr[pl.ds(0, LANES)] * wb
                return cy
            lax.fori_loop(j * (EPB // LANES), (j + 1) * (EPB // LANES),
                          scale16, 0)

            pltpu.sync_copy(rows_v.at[pl.ds(j * EPB, EPB)],
                            acc.at[dst_v.at[j]], add=True)
        return carry

    lax.fori_loop(0, CHUNKS_PER_TILE, chunk_body, 0)

    plsc.subcore_barrier()

    @pl.when(jnp.logical_and(c == 0, s < N_TILES - 1))
    def _():
        pltpu.sync_copy(acc.at[pl.ds(s * ACC_BLK, ACC_BLK)],
                        lo_hbm.at[pl.ds(s * ACC_BLK, ACC_BLK)])

    @pl.when(jnp.logical_and(c == 0, s == N_TILES - 1))
    def _():
        pltpu.sync_copy(acc.at[pl.ds((N_TILES - 1) * ACC_BLK, ACC_LAST)],
                        lo_hbm.at[pl.ds((N_TILES - 1) * ACC_BLK, ACC_LAST)])

    @pl.when(jnp.logical_and(c == 1, s < N_TILES - 1))
    def _():
        pltpu.sync_copy(acc.at[pl.ds(s * ACC_BLK, ACC_BLK)],
                        hi_hbm.at[pl.ds(s * ACC_BLK, ACC_BLK)])

    @pl.when(jnp.logical_and(c == 1, s == N_TILES - 1))
    def _():
        pltpu.sync_copy(acc.at[pl.ds((N_TILES - 1) * ACC_BLK, ACC_LAST)],
                        hi_hbm.at[pl.ds((N_TILES - 1) * ACC_BLK, ACC_LAST)])


_propagate = functools.partial(
    pl.kernel,
    out_type=[
        jax.ShapeDtypeStruct((N_NODES, HDIM), jnp.float32),
        jax.ShapeDtypeStruct((N_NODES, HDIM), jnp.float32),
    ],
    mesh=plsc.VectorSubcoreMesh(core_axis_name="c", subcore_axis_name="s"),
    compiler_params=pltpu.CompilerParams(use_tc_tiling_on_sc=False),
    scratch_types=[
        pltpu.VMEM((ROWS_PER_CHUNK, EPB), jnp.int32),
        pltpu.VMEM((ROWS_PER_CHUNK, EPB), jnp.int32),
        pltpu.VMEM((CHUNK_E,), jnp.float32),
        pltpu.VMEM((CHUNK_E, HDIM), jnp.float32),
        pltpu.VMEM_SHARED((ACC_ROWS, HDIM), jnp.float32),
        pltpu.SemaphoreType.DMA,
    ],
)(_prop_body)


def _mean4_body(a_ref, b_ref, c_ref, d_ref, o_ref):
    o_ref[...] = (a_ref[...] + b_ref[...] + c_ref[...] + d_ref[...]) * 0.25


def _mean4(x0, x1, x2, x3):
    n_flat_rows = N_NODES * DIM // 128
    blk = 1000
    xs = [x.reshape(n_flat_rows, 128) for x in (x0, x1, x2, x3)]
    total = pl.pallas_call(
        _mean4_body,
        out_shape=jax.ShapeDtypeStruct((n_flat_rows, 128), jnp.float32),
        grid=(n_flat_rows // blk,),
        in_specs=[pl.BlockSpec((blk, 128), lambda i: (i, 0))] * 4,
        out_specs=pl.BlockSpec((blk, 128), lambda i: (i, 0)),
    )(*xs)
    return total.reshape(N_NODES, DIM)


def _interleave(lo, hi):
    return jnp.stack([lo, hi], axis=1).reshape(N2, HDIM)


def kernel(user_emb, item_emb, edge_index, edge_weight):
    x0 = jnp.concatenate([user_emb, item_emb], axis=0)
    src = edge_index[0].astype(jnp.int32)
    dst = edge_index[1].astype(jnp.int32)
    w = edge_weight.astype(jnp.float32)
    pad = E_PAD - N_EDGES
    srcp = jnp.pad(src, (0, pad)).reshape(IDX_ROWS, EPB)
    dstp = jnp.pad(dst, (0, pad)).reshape(IDX_ROWS, EPB)
    wp = jnp.pad(w, (0, pad))
    zblk = jnp.zeros((ACC_BLK, HDIM), jnp.float32)

    xi0 = x0.reshape(N2, HDIM)
    lo1, hi1 = _propagate(xi0, srcp, dstp, wp, zblk)
    xi1 = _interleave(lo1, hi1)
    lo2, hi2 = _propagate(xi1, srcp, dstp, wp, zblk)
    xi2 = _interleave(lo2, hi2)
    lo3, hi3 = _propagate(xi2, srcp, dstp, wp, zblk)
    xi3 = _interleave(lo3, hi3)
    total = _mean4(xi0, xi1, xi2, xi3)
    return total[:USER_COUNT], total[USER_COUNT:]


# speedup vs baseline: 12.9083x; 1.0230x over previous
import functools

import jax
import jax.numpy as jnp
from jax import lax
from jax.experimental import pallas as pl
from jax.experimental.pallas import tpu as pltpu
from jax.experimental.pallas import tpu_sc as plsc

USER_COUNT = 50000
ITEM_COUNT = 50000
N_NODES = USER_COUNT + ITEM_COUNT
N_EDGES = 1600000
DIM = 32
N_LAYERS = 3

LANES = 16
HDIM = 16
N2 = 2 * N_NODES
EPB = 128
ROWS_PER_CHUNK = 10
CHUNK_E = EPB * ROWS_PER_CHUNK
N_TILES = 16
E_PAD = 1638400
IDX_ROWS = E_PAD // EPB
ROWS_PER_TILE = IDX_ROWS // N_TILES
CHUNKS_PER_TILE = ROWS_PER_TILE // ROWS_PER_CHUNK
ACC_BLK = 6256
ACC_ROWS = ACC_BLK * N_TILES
ACC_LAST = N_NODES - ACC_BLK * (N_TILES - 1)


_GDN = lax.GatherDimensionNumbers(
    offset_dims=(), collapsed_slice_dims=(0,), start_index_map=(0,))


def _bcast_lane(vec, lane):
    idx = jnp.full((LANES, 1), lane, jnp.int32)
    return lax.gather(vec, idx, _GDN, (1,),
                      mode=lax.GatherScatterMode.PROMISE_IN_BOUNDS)


def _prop_body(x_hbm, src_hbm, dst_hbm, w_hbm, zero_hbm, lo_hbm, hi_hbm,
               src_v, dst_v, w_v, rows_v, acc, gsem):
    c = lax.axis_index("c")
    s = lax.axis_index("s")

    pltpu.sync_copy(zero_hbm, acc.at[pl.ds(s * ACC_BLK, ACC_BLK)])

    plsc.subcore_barrier()

    def chunk_body(k, carry):
        row0 = s * ROWS_PER_TILE + k * ROWS_PER_CHUNK
        e0 = row0 * EPB
        pltpu.sync_copy(src_hbm.at[pl.ds(row0, ROWS_PER_CHUNK)], src_v)
        pltpu.sync_copy(dst_hbm.at[pl.ds(row0, ROWS_PER_CHUNK)], dst_v)
        pltpu.sync_copy(w_hbm.at[pl.ds(e0, CHUNK_E)], w_v)

        def adj_row(j, cy):
            sr = src_v.at[j]
            for i in range(EPB // LANES):
                sv = sr[pl.ds(i * LANES, LANES)]
                sr[pl.ds(i * LANES, LANES)] = sv + sv + c
            return cy
        lax.fori_loop(0, ROWS_PER_CHUNK, adj_row, 0)

        cps = [pltpu.async_copy(x_hbm.at[src_v.at[j]],
                                rows_v.at[pl.ds(j * EPB, EPB)], gsem)
               for j in range(ROWS_PER_CHUNK)]

        for j in range(ROWS_PER_CHUNK):
            cps[j].wait()

            def scale16(g, cy):
                wv = w_v[pl.ds(g * LANES, LANES)]
                for u in range(LANES):
                    wb = _bcast_lane(wv, u)
                    rr = rows_v.at[g * LANES + u]
                    rr[pl.ds(0, LANES)] = rr[pl.ds(0, LANES)] * wb
                return cy
            lax.fori_loop(j * (EPB // LANES), (j + 1) * (EPB // LANES),
                          scale16, 0)

            pltpu.sync_copy(rows_v.at[pl.ds(j * EPB, EPB)],
                            acc.at[dst_v.at[j]], add=True)
        return carry

    lax.fori_loop(0, CHUNKS_PER_TILE, chunk_body, 0)

    plsc.subcore_barrier()

    @pl.when(jnp.logical_and(c == 0, s < N_TILES - 1))
    def _():
        pltpu.sync_copy(acc.at[pl.ds(s * ACC_BLK, ACC_BLK)],
                        lo_hbm.at[pl.ds(s * ACC_BLK, ACC_BLK)])

    @pl.when(jnp.logical_and(c == 0, s == N_TILES - 1))
    def _():
        pltpu.sync_copy(acc.at[pl.ds((N_TILES - 1) * ACC_BLK, ACC_LAST)],
                        lo_hbm.at[pl.ds((N_TILES - 1) * ACC_BLK, ACC_LAST)])

    @pl.when(jnp.logical_and(c == 1, s < N_TILES - 1))
    def _():
        pltpu.sync_copy(acc.at[pl.ds(s * ACC_BLK, ACC_BLK)],
                        hi_hbm.at[pl.ds(s * ACC_BLK, ACC_BLK)])

    @pl.when(jnp.logical_and(c == 1, s == N_TILES - 1))
    def _():
        pltpu.sync_copy(acc.at[pl.ds((N_TILES - 1) * ACC_BLK, ACC_LAST)],
                        hi_hbm.at[pl.ds((N_TILES - 1) * ACC_BLK, ACC_LAST)])


_propagate = functools.partial(
    pl.kernel,
    out_type=[
        jax.ShapeDtypeStruct((N_NODES, HDIM), jnp.float32),
        jax.ShapeDtypeStruct((N_NODES, HDIM), jnp.float32),
    ],
    mesh=plsc.VectorSubcoreMesh(core_axis_name="c", subcore_axis_name="s"),
    compiler_params=pltpu.CompilerParams(use_tc_tiling_on_sc=False),
    scratch_types=[
        pltpu.VMEM((ROWS_PER_CHUNK, EPB), jnp.int32),
        pltpu.VMEM((ROWS_PER_CHUNK, EPB), jnp.int32),
        pltpu.VMEM((CHUNK_E,), jnp.float32),
        pltpu.VMEM((CHUNK_E, HDIM), jnp.float32),
        pltpu.VMEM_SHARED((ACC_ROWS, HDIM), jnp.float32),
        pltpu.SemaphoreType.DMA,
    ],
)(_prop_body)


def _mean4_body(a_ref, b_ref, c_ref, d_ref, o_ref):
    o_ref[...] = (a_ref[...] + b_ref[...] + c_ref[...] + d_ref[...]) * 0.25


def _mean4(x0, x1, x2, x3):
    n_flat_rows = N_NODES * DIM // 128
    blk = 1000
    xs = [x.reshape(n_flat_rows, 128) for x in (x0, x1, x2, x3)]
    total = pl.pallas_call(
        _mean4_body,
        out_shape=jax.ShapeDtypeStruct((n_flat_rows, 128), jnp.float32),
        grid=(n_flat_rows // blk,),
        in_specs=[pl.BlockSpec((blk, 128), lambda i: (i, 0))] * 4,
        out_specs=pl.BlockSpec((blk, 128), lambda i: (i, 0)),
    )(*xs)
    return total.reshape(N_NODES, DIM)


def _interleave(lo, hi):
    return jnp.stack([lo, hi], axis=1).reshape(N2, HDIM)


def kernel(user_emb, item_emb, edge_index, edge_weight):
    x0 = jnp.concatenate([user_emb, item_emb], axis=0)
    src = edge_index[0].astype(jnp.int32)
    dst = edge_index[1].astype(jnp.int32)
    w = edge_weight.astype(jnp.float32)
    pad = E_PAD - N_EDGES
    srcp = jnp.pad(src, (0, pad)).reshape(IDX_ROWS, EPB)
    dstp = jnp.pad(dst, (0, pad)).reshape(IDX_ROWS, EPB)
    wp = jnp.pad(w, (0, pad))
    zblk = jnp.zeros((ACC_BLK, HDIM), jnp.float32)

    xi0 = x0.reshape(N2, HDIM)
    lo1, hi1 = _propagate(xi0, srcp, dstp, wp, zblk)
    xi1 = _interleave(lo1, hi1)
    lo2, hi2 = _propagate(xi1, srcp, dstp, wp, zblk)
    xi2 = _interleave(lo2, hi2)
    lo3, hi3 = _propagate(xi2, srcp, dstp, wp, zblk)
    xi3 = _interleave(lo3, hi3)
    total = _mean4(xi0, xi1, xi2, xi3)
    return total[:USER_COUNT], total[USER_COUNT:]
